# baseline (speedup 1.0000x reference)
import functools

import jax
import jax.numpy as jnp
from jax import lax
from jax.experimental import pallas as pl
from jax.experimental.pallas import tpu as pltpu
from jax.experimental.pallas import tpu_sc as plsc

B = 16384
D = 64
NC = 2
NS = 16
NW = NC * NS
BPW = B // NW
CH = 128
NCH = BPW // CH
RB = 1024


def _sc_body(head_hbm, rel_hbm, tail_hbm, ent_hbm, relemb_hbm,
             oh_hbm, or_hbm, ot_hbm, hidx, ridx, tidx, hrow, rrow, trow, sem):
    wid = lax.axis_index("s") * NC + lax.axis_index("c")
    base = wid * BPW
    rowblk = wid * NCH

    pltpu.sync_copy(head_hbm.at[pl.ds(rowblk, NCH)], hidx)
    pltpu.sync_copy(rel_hbm.at[pl.ds(rowblk, NCH)], ridx)
    pltpu.sync_copy(tail_hbm.at[pl.ds(rowblk, NCH)], tidx)

    copies = []
    for j in range(NCH):
        rows = pl.ds(j * CH, CH)
        copies.append(pltpu.async_copy(
            ent_hbm.at[hidx.at[j]], hrow.at[rows], sem))
        copies.append(pltpu.async_copy(
            relemb_hbm.at[ridx.at[j]], rrow.at[rows], sem))
        copies.append(pltpu.async_copy(
            ent_hbm.at[tidx.at[j]], trow.at[rows], sem))
    for c in copies:
        c.wait()

    pltpu.sync_copy(hrow, oh_hbm.at[pl.ds(base, BPW)])
    pltpu.sync_copy(rrow, or_hbm.at[pl.ds(base, BPW)])
    pltpu.sync_copy(trow, ot_hbm.at[pl.ds(base, BPW)])


def _concat_body(h_ref, r_ref, t_ref, o_ref):
    o_ref[...] = jnp.concatenate([h_ref[...], r_ref[...], t_ref[...]], axis=-1)


@jax.jit
def _lookup(head2, rel2, tail2, ent_embeddings, rel_embeddings):
    gather = pl.kernel(
        _sc_body,
        mesh=plsc.VectorSubcoreMesh(core_axis_name="c", subcore_axis_name="s"),
        out_type=(
            jax.ShapeDtypeStruct((B, D), jnp.float32),
            jax.ShapeDtypeStruct((B, D), jnp.float32),
            jax.ShapeDtypeStruct((B, D), jnp.float32),
        ),
        scratch_types=[
            pltpu.VMEM((NCH, CH), jnp.int32),
            pltpu.VMEM((NCH, CH), jnp.int32),
            pltpu.VMEM((NCH, CH), jnp.int32),
            pltpu.VMEM((BPW, D), jnp.float32),
            pltpu.VMEM((BPW, D), jnp.float32),
            pltpu.VMEM((BPW, D), jnp.float32),
            pltpu.SemaphoreType.DMA,
        ],
        compiler_params=pltpu.CompilerParams(use_tc_tiling_on_sc=False),
    )
    h, r, t = gather(head2, rel2, tail2, ent_embeddings, rel_embeddings)

    concat = pl.pallas_call(
        _concat_body,
        grid=(B // RB,),
        in_specs=[pl.BlockSpec((RB, D), lambda i: (i, 0))] * 3,
        out_specs=pl.BlockSpec((RB, 3 * D), lambda i: (i, 0)),
        out_shape=jax.ShapeDtypeStruct((B, 3 * D), jnp.float32),
    )
    return concat(h, r, t)


def kernel(head, rel, tail, ent_embeddings, rel_embeddings):
    head2 = head.reshape(B // CH, CH)
    rel2 = rel.reshape(B // CH, CH)
    tail2 = tail.reshape(B // CH, CH)
    return _lookup(head2, rel2, tail2, ent_embeddings, rel_embeddings)
